# Initial kernel scaffold; baseline (speedup 1.0000x reference)
import jax
import jax.numpy as jnp
from jax.experimental import pallas as pl


def kernel(x, sources, targets, norm, W):
    raise NotImplementedError("write your pallas kernel here")


# speedup vs baseline: 4.2408x; 4.2408x over previous
import functools

import jax
import jax.numpy as jnp
from jax import lax
from jax.experimental import pallas as pl
from jax.experimental.pallas import tpu as pltpu
from jax.experimental.pallas import tpu_sc as plsc

N = 50000
E = 800000
C = 64

NC = 2
NS = 16
HALF = N // NC
DUMMY = HALF
ACC_ROWS = HALF + 8

EPS = E // NS
CHUNK = 128
NFULL = EPS // CHUNK
REM = EPS - NFULL * CHUNK

INIT_SZ = 1568
INIT_LAST = HALF - (NS - 1) * INIT_SZ

ROWBLK = 1000


def _sc_body(x_hbm, src_hbm, tgt_hbm, agg_hbm,
             acc, rows, sidx, tstage, tidx, sidx_r, tstage_r, tidx_r, sem):
    c = lax.axis_index("c")
    s = lax.axis_index("s")
    base_node = c * HALF

    @pl.when(s < NS - 1)
    def _():
        pltpu.sync_copy(x_hbm.at[pl.ds(base_node + s * INIT_SZ, INIT_SZ)],
                        acc.at[pl.ds(s * INIT_SZ, INIT_SZ)])

    @pl.when(s == NS - 1)
    def _():
        pltpu.sync_copy(x_hbm.at[pl.ds(base_node + (NS - 1) * INIT_SZ, INIT_LAST)],
                        acc.at[pl.ds((NS - 1) * INIT_SZ, INIT_LAST)])

    plsc.subcore_barrier()

    e0 = s * EPS

    def _remap(src_ref, dst_ref, n):
        for i in range(n // 16):
            t = src_ref[pl.ds(i * 16, 16)]
            tl = t - base_node
            ok = (tl >= 0) & (tl < HALF)
            dst_ref[pl.ds(i * 16, 16)] = jnp.where(ok, tl, DUMMY)

    def chunk_body(j, carry):
        eb = e0 + j * CHUNK
        pltpu.sync_copy(src_hbm.at[pl.ds(eb, CHUNK)], sidx)
        pltpu.sync_copy(tgt_hbm.at[pl.ds(eb, CHUNK)], tstage)
        _remap(tstage, tidx, CHUNK)
        pltpu.async_copy(x_hbm.at[sidx], rows, sem).wait()
        pltpu.sync_copy(rows, acc.at[tidx], add=True)
        return carry

    lax.fori_loop(0, NFULL, chunk_body, 0)

    eb = e0 + NFULL * CHUNK
    pltpu.sync_copy(src_hbm.at[pl.ds(eb, REM)], sidx_r)
    pltpu.sync_copy(tgt_hbm.at[pl.ds(eb, REM)], tstage_r)
    _remap(tstage_r, tidx_r, REM)
    pltpu.async_copy(x_hbm.at[sidx_r], rows.at[pl.ds(0, REM)], sem).wait()
    pltpu.sync_copy(rows.at[pl.ds(0, REM)], acc.at[tidx_r], add=True)

    plsc.subcore_barrier()

    @pl.when(s < NS - 1)
    def _():
        pltpu.sync_copy(acc.at[pl.ds(s * INIT_SZ, INIT_SZ)],
                        agg_hbm.at[pl.ds(base_node + s * INIT_SZ, INIT_SZ)])

    @pl.when(s == NS - 1)
    def _():
        pltpu.sync_copy(acc.at[pl.ds((NS - 1) * INIT_SZ, INIT_LAST)],
                        agg_hbm.at[pl.ds(base_node + (NS - 1) * INIT_SZ, INIT_LAST)])


_sc_aggregate = functools.partial(
    pl.kernel,
    out_type=jax.ShapeDtypeStruct((N, C), jnp.float32),
    mesh=plsc.VectorSubcoreMesh(core_axis_name="c", subcore_axis_name="s"),
    compiler_params=pltpu.CompilerParams(use_tc_tiling_on_sc=False),
    scratch_types=[
        pltpu.VMEM_SHARED((ACC_ROWS, C), jnp.float32),
        pltpu.VMEM((CHUNK, C), jnp.float32),
        pltpu.VMEM((CHUNK,), jnp.int32),
        pltpu.VMEM((CHUNK,), jnp.int32),
        pltpu.VMEM((CHUNK,), jnp.int32),
        pltpu.VMEM((REM,), jnp.int32),
        pltpu.VMEM((REM,), jnp.int32),
        pltpu.VMEM((REM,), jnp.int32),
        pltpu.SemaphoreType.DMA,
    ],
)(_sc_body)


def _tc_body(norm_ref, agg_ref, w_ref, out_ref):
    h = norm_ref[...] * agg_ref[...]
    out_ref[...] = jnp.dot(h, w_ref[...], preferred_element_type=jnp.float32)


def _tc_matmul(norm, agg, W):
    return pl.pallas_call(
        _tc_body,
        grid=(N // ROWBLK,),
        in_specs=[
            pl.BlockSpec((ROWBLK, 1), lambda i: (i, 0)),
            pl.BlockSpec((ROWBLK, C), lambda i: (i, 0)),
            pl.BlockSpec((C, C), lambda i: (0, 0)),
        ],
        out_specs=pl.BlockSpec((ROWBLK, C), lambda i: (i, 0)),
        out_shape=jax.ShapeDtypeStruct((N, C), jnp.float32),
    )(norm, agg, W)


def kernel(x, sources, targets, norm, W):
    src = sources.astype(jnp.int32)
    tgt = targets.astype(jnp.int32)
    agg = _sc_aggregate(x, src, tgt)
    return _tc_matmul(norm, agg, W)


# speedup vs baseline: 8.4333x; 1.9886x over previous
import functools

import jax
import jax.numpy as jnp
from jax import lax
from jax.experimental import pallas as pl
from jax.experimental.pallas import tpu as pltpu
from jax.experimental.pallas import tpu_sc as plsc

N = 50000
E = 800000
C = 64

NC = 2
NS = 16
HALF = N // NC
ACC_ROWS = HALF + NS

EPS = E // NS
CHUNK = 128
IDXBLK = 1024
CPB = IDXBLK // CHUNK
NBLK = EPS // IDXBLK
REMC = (EPS - NBLK * IDXBLK) // CHUNK
TAIL = EPS - NBLK * IDXBLK - REMC * CHUNK

INIT_SZ = 1568
INIT_LAST = HALF - (NS - 1) * INIT_SZ

ROWBLK = 1000


def _sc_body(x_hbm, src_hbm, tgt_hbm, agg_hbm,
             acc, rows0, rows1, sblk, tblk, tidxblk, tidx_t,
             sem0, sem1):
    c = lax.axis_index("c")
    s = lax.axis_index("s")
    base_node = c * HALF
    dummy = HALF + s

    @pl.when(s < NS - 1)
    def _():
        pltpu.sync_copy(x_hbm.at[pl.ds(base_node + s * INIT_SZ, INIT_SZ)],
                        acc.at[pl.ds(s * INIT_SZ, INIT_SZ)])

    @pl.when(s == NS - 1)
    def _():
        pltpu.sync_copy(x_hbm.at[pl.ds(base_node + (NS - 1) * INIT_SZ, INIT_LAST)],
                        acc.at[pl.ds((NS - 1) * INIT_SZ, INIT_LAST)])

    plsc.subcore_barrier()

    e0 = s * EPS
    bufs = (rows0, rows1)
    sems = (sem0, sem1)

    def process_block(eb, nch):
        blen = nch * CHUNK
        pltpu.sync_copy(src_hbm.at[pl.ds(eb, blen)], sblk.at[pl.ds(0, blen)])
        pltpu.sync_copy(tgt_hbm.at[pl.ds(eb, blen)], tblk.at[pl.ds(0, blen)])
        for kk in range(nch):
            for ii in range(CHUNK // 16):
                i = kk * (CHUNK // 16) + ii
                t = tblk[pl.ds(i * 16, 16)]
                tl = t - base_node
                ok = (tl >= 0) & (tl < HALF)
                tidxblk[kk, pl.ds(ii * 16, 16)] = jnp.where(ok, tl, dummy)
        descs = [None] * nch
        descs[0] = pltpu.async_copy(
            x_hbm.at[sblk.at[pl.ds(0, CHUNK)]], bufs[0], sems[0])
        for kk in range(nch):
            nk = kk + 1
            if nk < nch:
                descs[nk] = pltpu.async_copy(
                    x_hbm.at[sblk.at[pl.ds(nk * CHUNK, CHUNK)]],
                    bufs[nk % 2], sems[nk % 2])
            descs[kk].wait()
            pltpu.sync_copy(bufs[kk % 2], acc.at[tidxblk.at[kk]], add=True)

    lax.fori_loop(0, NBLK, lambda k, cr: (process_block(e0 + k * IDXBLK, CPB), cr)[1], 0)
    process_block(e0 + NBLK * IDXBLK, REMC)

    et = e0 + NBLK * IDXBLK + REMC * CHUNK
    pltpu.sync_copy(src_hbm.at[pl.ds(et, TAIL)], sblk.at[pl.ds(0, TAIL)])
    pltpu.sync_copy(tgt_hbm.at[pl.ds(et, TAIL)], tblk.at[pl.ds(0, TAIL)])
    for ii in range(TAIL // 16):
        t = tblk[pl.ds(ii * 16, 16)]
        tl = t - base_node
        ok = (tl >= 0) & (tl < HALF)
        tidx_t[pl.ds(ii * 16, 16)] = jnp.where(ok, tl, dummy)
    pltpu.async_copy(x_hbm.at[sblk.at[pl.ds(0, TAIL)]],
                     rows0.at[pl.ds(0, TAIL)], sem0).wait()
    pltpu.sync_copy(rows0.at[pl.ds(0, TAIL)], acc.at[tidx_t], add=True)

    plsc.subcore_barrier()

    @pl.when(s < NS - 1)
    def _():
        pltpu.sync_copy(acc.at[pl.ds(s * INIT_SZ, INIT_SZ)],
                        agg_hbm.at[pl.ds(base_node + s * INIT_SZ, INIT_SZ)])

    @pl.when(s == NS - 1)
    def _():
        pltpu.sync_copy(acc.at[pl.ds((NS - 1) * INIT_SZ, INIT_LAST)],
                        agg_hbm.at[pl.ds(base_node + (NS - 1) * INIT_SZ, INIT_LAST)])


_sc_aggregate = functools.partial(
    pl.kernel,
    out_type=jax.ShapeDtypeStruct((N, C), jnp.float32),
    mesh=plsc.VectorSubcoreMesh(core_axis_name="c", subcore_axis_name="s"),
    compiler_params=pltpu.CompilerParams(use_tc_tiling_on_sc=False),
    scratch_types=[
        pltpu.VMEM_SHARED((ACC_ROWS, C), jnp.float32),
        pltpu.VMEM((CHUNK, C), jnp.float32),
        pltpu.VMEM((CHUNK, C), jnp.float32),
        pltpu.VMEM((IDXBLK,), jnp.int32),
        pltpu.VMEM((IDXBLK,), jnp.int32),
        pltpu.VMEM((CPB, CHUNK), jnp.int32),
        pltpu.VMEM((TAIL,), jnp.int32),
        pltpu.SemaphoreType.DMA,
        pltpu.SemaphoreType.DMA,
    ],
)(_sc_body)


def _tc_body(norm_ref, agg_ref, w_ref, out_ref):
    h = norm_ref[...] * agg_ref[...]
    out_ref[...] = jnp.dot(h, w_ref[...], preferred_element_type=jnp.float32)


def _tc_matmul(norm, agg, W):
    return pl.pallas_call(
        _tc_body,
        grid=(N // ROWBLK,),
        in_specs=[
            pl.BlockSpec((ROWBLK, 1), lambda i: (i, 0)),
            pl.BlockSpec((ROWBLK, C), lambda i: (i, 0)),
            pl.BlockSpec((C, C), lambda i: (0, 0)),
        ],
        out_specs=pl.BlockSpec((ROWBLK, C), lambda i: (i, 0)),
        out_shape=jax.ShapeDtypeStruct((N, C), jnp.float32),
    )(norm, agg, W)


def kernel(x, sources, targets, norm, W):
    src = sources.astype(jnp.int32)
    tgt = targets.astype(jnp.int32)
    agg = _sc_aggregate(x, src, tgt)
    return _tc_matmul(norm, agg, W)


# speedup vs baseline: 10.6190x; 1.2592x over previous
import functools

import jax
import jax.numpy as jnp
from jax import lax
from jax.experimental import pallas as pl
from jax.experimental.pallas import tpu as pltpu
from jax.experimental.pallas import tpu_sc as plsc

N = 50000
E = 800000
C = 64

NC = 2
NS = 16
HALF = N // NC
ACC_ROWS = HALF + NS

EPS = E // NS
CHUNK = 96
CPB = 4
IDXBLK = CPB * CHUNK
NBLK = EPS // IDXBLK
REMC = (EPS - NBLK * IDXBLK) // CHUNK
TAIL = EPS - NBLK * IDXBLK - REMC * CHUNK
NBUF = 4
LOOKAHEAD = 2

INIT_SZ = 1568
INIT_LAST = HALF - (NS - 1) * INIT_SZ

ROWBLK = 1000


def _sc_body(x_hbm, src_hbm, tgt_hbm, agg_hbm,
             acc, rowbufs, sblk, tblk, tidx, tidx_t,
             isem, gsems, ssems):
    c = lax.axis_index("c")
    s = lax.axis_index("s")
    base_node = c * HALF
    dummy = HALF + s

    @pl.when(s < NS - 1)
    def _():
        pltpu.sync_copy(x_hbm.at[pl.ds(base_node + s * INIT_SZ, INIT_SZ)],
                        acc.at[pl.ds(s * INIT_SZ, INIT_SZ)])

    @pl.when(s == NS - 1)
    def _():
        pltpu.sync_copy(x_hbm.at[pl.ds(base_node + (NS - 1) * INIT_SZ, INIT_LAST)],
                        acc.at[pl.ds((NS - 1) * INIT_SZ, INIT_LAST)])

    plsc.subcore_barrier()

    e0 = s * EPS

    def remap(tsrc, tdst, nch):
        for kk in range(nch):
            for ii in range(CHUNK // 16):
                i = kk * (CHUNK // 16) + ii
                t = tsrc[pl.ds(i * 16, 16)]
                tl = t - base_node
                ok = (tl >= 0) & (tl < HALF)
                tdst[kk, pl.ds(ii * 16, 16)] = jnp.where(ok, tl, dummy)

    def fire_gather(p, j):
        return pltpu.async_copy(
            x_hbm.at[sblk.at[p].at[pl.ds(j * CHUNK, CHUNK)]],
            rowbufs[j], gsems[j])

    def fire_scatter(p, j):
        return pltpu.async_copy(
            rowbufs[j], acc.at[tidx.at[p].at[j]], ssems[j], add=True)

    def drain_scatter(p, j):
        pltpu.make_async_copy(
            rowbufs[j], acc.at[tidx.at[p].at[j]], ssems[j]).wait()

    def load_idx(p, blk):
        eb = e0 + blk * IDXBLK
        pltpu.async_copy(src_hbm.at[pl.ds(eb, IDXBLK)], sblk.at[p], isem)
        pltpu.async_copy(tgt_hbm.at[pl.ds(eb, IDXBLK)], tblk.at[p], isem)

    def drain_idx(p):
        pltpu.make_async_copy(src_hbm.at[pl.ds(e0, IDXBLK)], sblk.at[p], isem).wait()
        pltpu.make_async_copy(tgt_hbm.at[pl.ds(e0, IDXBLK)], tblk.at[p], isem).wait()

    pltpu.sync_copy(src_hbm.at[pl.ds(e0, IDXBLK)], sblk.at[0])
    pltpu.sync_copy(tgt_hbm.at[pl.ds(e0, IDXBLK)], tblk.at[0])
    load_idx(1, 1)
    descs = [fire_gather(0, j) for j in range(LOOKAHEAD)]
    remap(tblk.at[0], tidx.at[0], CPB)
    for j in range(LOOKAHEAD, CPB):
        descs.append(fire_gather(0, j))
    for j in range(CPB):
        descs[j].wait()
        fire_scatter(0, j)
    drain_idx(1)
    for j in range(LOOKAHEAD):
        drain_scatter(0, j)
        fire_gather(1, j)

    def body(g, carry):
        p = g % 2
        q = 1 - p
        remap(tblk.at[p], tidx.at[p], CPB)

        @pl.when(g + 1 < NBLK)
        def _():
            load_idx(q, g + 1)

        ds2 = []
        for j in range(LOOKAHEAD, CPB):
            drain_scatter(q, j)
            ds2.append(fire_gather(p, j))
        for j in range(CPB):
            if j < LOOKAHEAD:
                pltpu.make_async_copy(
                    x_hbm.at[sblk.at[p].at[pl.ds(j * CHUNK, CHUNK)]],
                    rowbufs[j], gsems[j]).wait()
            else:
                ds2[j - LOOKAHEAD].wait()
            fire_scatter(p, j)

        @pl.when(g + 1 < NBLK)
        def _():
            drain_idx(q)
            for j in range(LOOKAHEAD):
                drain_scatter(p, j)
                fire_gather(q, j)

        return carry

    lax.fori_loop(1, NBLK, body, 0, unroll=False)

    p_last = (NBLK - 1) % 2
    eb = e0 + NBLK * IDXBLK
    if REMC > 0:
        pltpu.sync_copy(src_hbm.at[pl.ds(eb, REMC * CHUNK)],
                        sblk.at[p_last].at[pl.ds(0, REMC * CHUNK)])
        pltpu.sync_copy(tgt_hbm.at[pl.ds(eb, REMC * CHUNK)],
                        tblk.at[p_last].at[pl.ds(0, REMC * CHUNK)])
        remap(tblk.at[p_last], tidx.at[p_last], REMC)
        ds3 = []
        for j in range(REMC):
            drain_scatter(p_last, j)
            ds3.append(fire_gather(p_last, j))
        for j in range(REMC):
            ds3[j].wait()
            fire_scatter(p_last, j)

    et = eb + REMC * CHUNK
    pltpu.sync_copy(src_hbm.at[pl.ds(et, TAIL)],
                    sblk.at[p_last].at[pl.ds(0, TAIL)])
    pltpu.sync_copy(tgt_hbm.at[pl.ds(et, TAIL)],
                    tblk.at[p_last].at[pl.ds(0, TAIL)])
    for ii in range(TAIL // 16):
        t = tblk[p_last, pl.ds(ii * 16, 16)]
        tl = t - base_node
        ok = (tl >= 0) & (tl < HALF)
        tidx_t[pl.ds(ii * 16, 16)] = jnp.where(ok, tl, dummy)
    drain_scatter(p_last, REMC)
    pltpu.async_copy(
        x_hbm.at[sblk.at[p_last].at[pl.ds(0, TAIL)]],
        rowbufs[REMC].at[pl.ds(0, TAIL)], gsems[REMC]).wait()
    pltpu.sync_copy(rowbufs[REMC].at[pl.ds(0, TAIL)], acc.at[tidx_t], add=True)

    for j in range(REMC):
        drain_scatter(p_last, j)
    for j in range(REMC + 1, NBUF):
        drain_scatter(p_last, j)

    plsc.subcore_barrier()

    @pl.when(s < NS - 1)
    def _():
        pltpu.sync_copy(acc.at[pl.ds(s * INIT_SZ, INIT_SZ)],
                        agg_hbm.at[pl.ds(base_node + s * INIT_SZ, INIT_SZ)])

    @pl.when(s == NS - 1)
    def _():
        pltpu.sync_copy(acc.at[pl.ds((NS - 1) * INIT_SZ, INIT_LAST)],
                        agg_hbm.at[pl.ds(base_node + (NS - 1) * INIT_SZ, INIT_LAST)])


_sc_aggregate = functools.partial(
    pl.kernel,
    out_type=jax.ShapeDtypeStruct((N, C), jnp.float32),
    mesh=plsc.VectorSubcoreMesh(core_axis_name="c", subcore_axis_name="s"),
    compiler_params=pltpu.CompilerParams(use_tc_tiling_on_sc=False),
    scratch_types=[
        pltpu.VMEM_SHARED((ACC_ROWS, C), jnp.float32),
        [pltpu.VMEM((CHUNK, C), jnp.float32)] * NBUF,
        pltpu.VMEM((2, IDXBLK), jnp.int32),
        pltpu.VMEM((2, IDXBLK), jnp.int32),
        pltpu.VMEM((2, CPB, CHUNK), jnp.int32),
        pltpu.VMEM((TAIL,), jnp.int32),
        pltpu.SemaphoreType.DMA,
        [pltpu.SemaphoreType.DMA] * NBUF,
        [pltpu.SemaphoreType.DMA] * NBUF,
    ],
)(_sc_body)


def _tc_body(norm_ref, agg_ref, w_ref, out_ref):
    h = norm_ref[...] * agg_ref[...]
    out_ref[...] = jnp.dot(h, w_ref[...], preferred_element_type=jnp.float32)


def _tc_matmul(norm, agg, W):
    return pl.pallas_call(
        _tc_body,
        grid=(N // ROWBLK,),
        in_specs=[
            pl.BlockSpec((ROWBLK, 1), lambda i: (i, 0)),
            pl.BlockSpec((ROWBLK, C), lambda i: (i, 0)),
            pl.BlockSpec((C, C), lambda i: (0, 0)),
        ],
        out_specs=pl.BlockSpec((ROWBLK, C), lambda i: (i, 0)),
        out_shape=jax.ShapeDtypeStruct((N, C), jnp.float32),
    )(norm, agg, W)


def kernel(x, sources, targets, norm, W):
    src = sources.astype(jnp.int32)
    tgt = targets.astype(jnp.int32)
    agg = _sc_aggregate(x, src, tgt)
    return _tc_matmul(norm, agg, W)


# speedup vs baseline: 11.1919x; 1.0539x over previous
import functools

import jax
import jax.numpy as jnp
from jax import lax
from jax.experimental import pallas as pl
from jax.experimental.pallas import tpu as pltpu
from jax.experimental.pallas import tpu_sc as plsc

N = 50000
E = 800000
C = 64

NC = 2
NS = 16
HALF = N // NC
ACC_ROWS = HALF + NS

EPS = E // NS
CHUNK = 96
CPB = 4
IDXBLK = CPB * CHUNK
NBLK = EPS // IDXBLK
REMC = (EPS - NBLK * IDXBLK) // CHUNK
TAIL = EPS - NBLK * IDXBLK - REMC * CHUNK
NBUF = 4
LOOKAHEAD = 2

INIT_SZ = 1568
INIT_LAST = HALF - (NS - 1) * INIT_SZ

ROWBLK = 5000


def _sc_body(x_hbm, src_hbm, tgt_hbm, agg_hbm,
             acc, rowbufs, sblk, tblk, tidx, tidx_t,
             isem, gsems, ssems):
    c = lax.axis_index("c")
    s = lax.axis_index("s")
    base_node = c * HALF
    dummy = HALF + s

    @pl.when(s < NS - 1)
    def _():
        pltpu.sync_copy(x_hbm.at[pl.ds(base_node + s * INIT_SZ, INIT_SZ)],
                        acc.at[pl.ds(s * INIT_SZ, INIT_SZ)])

    @pl.when(s == NS - 1)
    def _():
        pltpu.sync_copy(x_hbm.at[pl.ds(base_node + (NS - 1) * INIT_SZ, INIT_LAST)],
                        acc.at[pl.ds((NS - 1) * INIT_SZ, INIT_LAST)])

    plsc.subcore_barrier()

    e0 = s * EPS

    def remap(tsrc, tdst, nch):
        for kk in range(nch):
            for ii in range(CHUNK // 16):
                i = kk * (CHUNK // 16) + ii
                t = tsrc[pl.ds(i * 16, 16)]
                tl = t - base_node
                ok = (tl >= 0) & (tl < HALF)
                tdst[kk, pl.ds(ii * 16, 16)] = jnp.where(ok, tl, dummy)

    def fire_gather(p, j):
        return pltpu.async_copy(
            x_hbm.at[sblk.at[p].at[pl.ds(j * CHUNK, CHUNK)]],
            rowbufs[j], gsems[j])

    def fire_scatter(p, j):
        return pltpu.async_copy(
            rowbufs[j], acc.at[tidx.at[p].at[j]], ssems[j], add=True)

    def drain_scatter(p, j):
        pltpu.make_async_copy(
            rowbufs[j], acc.at[tidx.at[p].at[j]], ssems[j]).wait()

    def load_idx(p, blk):
        eb = e0 + blk * IDXBLK
        pltpu.async_copy(src_hbm.at[pl.ds(eb, IDXBLK)], sblk.at[p], isem)
        pltpu.async_copy(tgt_hbm.at[pl.ds(eb, IDXBLK)], tblk.at[p], isem)

    def drain_idx(p):
        pltpu.make_async_copy(src_hbm.at[pl.ds(e0, IDXBLK)], sblk.at[p], isem).wait()
        pltpu.make_async_copy(tgt_hbm.at[pl.ds(e0, IDXBLK)], tblk.at[p], isem).wait()

    pltpu.sync_copy(src_hbm.at[pl.ds(e0, IDXBLK)], sblk.at[0])
    pltpu.sync_copy(tgt_hbm.at[pl.ds(e0, IDXBLK)], tblk.at[0])
    load_idx(1, 1)
    descs = [fire_gather(0, j) for j in range(LOOKAHEAD)]
    remap(tblk.at[0], tidx.at[0], CPB)
    for j in range(LOOKAHEAD, CPB):
        descs.append(fire_gather(0, j))
    for j in range(CPB):
        descs[j].wait()
        fire_scatter(0, j)
    drain_idx(1)
    for j in range(LOOKAHEAD):
        drain_scatter(0, j)
        fire_gather(1, j)

    def body(g, carry):
        p = g % 2
        q = 1 - p
        remap(tblk.at[p], tidx.at[p], CPB)

        @pl.when(g + 1 < NBLK)
        def _():
            load_idx(q, g + 1)

        ds2 = []
        for j in range(LOOKAHEAD, CPB):
            drain_scatter(q, j)
            ds2.append(fire_gather(p, j))
        for j in range(CPB):
            if j < LOOKAHEAD:
                pltpu.make_async_copy(
                    x_hbm.at[sblk.at[p].at[pl.ds(j * CHUNK, CHUNK)]],
                    rowbufs[j], gsems[j]).wait()
            else:
                ds2[j - LOOKAHEAD].wait()
            fire_scatter(p, j)

        @pl.when(g + 1 < NBLK)
        def _():
            drain_idx(q)
            for j in range(LOOKAHEAD):
                drain_scatter(p, j)
                fire_gather(q, j)

        return carry

    lax.fori_loop(1, NBLK, body, 0, unroll=False)

    p_last = (NBLK - 1) % 2
    eb = e0 + NBLK * IDXBLK
    if REMC > 0:
        pltpu.sync_copy(src_hbm.at[pl.ds(eb, REMC * CHUNK)],
                        sblk.at[p_last].at[pl.ds(0, REMC * CHUNK)])
        pltpu.sync_copy(tgt_hbm.at[pl.ds(eb, REMC * CHUNK)],
                        tblk.at[p_last].at[pl.ds(0, REMC * CHUNK)])
        remap(tblk.at[p_last], tidx.at[p_last], REMC)
        ds3 = []
        for j in range(REMC):
            drain_scatter(p_last, j)
            ds3.append(fire_gather(p_last, j))
        for j in range(REMC):
            ds3[j].wait()
            fire_scatter(p_last, j)

    et = eb + REMC * CHUNK
    pltpu.sync_copy(src_hbm.at[pl.ds(et, TAIL)],
                    sblk.at[p_last].at[pl.ds(0, TAIL)])
    pltpu.sync_copy(tgt_hbm.at[pl.ds(et, TAIL)],
                    tblk.at[p_last].at[pl.ds(0, TAIL)])
    for ii in range(TAIL // 16):
        t = tblk[p_last, pl.ds(ii * 16, 16)]
        tl = t - base_node
        ok = (tl >= 0) & (tl < HALF)
        tidx_t[pl.ds(ii * 16, 16)] = jnp.where(ok, tl, dummy)
    drain_scatter(p_last, REMC)
    pltpu.async_copy(
        x_hbm.at[sblk.at[p_last].at[pl.ds(0, TAIL)]],
        rowbufs[REMC].at[pl.ds(0, TAIL)], gsems[REMC]).wait()
    pltpu.sync_copy(rowbufs[REMC].at[pl.ds(0, TAIL)], acc.at[tidx_t], add=True)

    for j in range(REMC):
        drain_scatter(p_last, j)
    for j in range(REMC + 1, NBUF):
        drain_scatter(p_last, j)

    plsc.subcore_barrier()

    @pl.when(s < NS - 1)
    def _():
        pltpu.sync_copy(acc.at[pl.ds(s * INIT_SZ, INIT_SZ)],
                        agg_hbm.at[pl.ds(base_node + s * INIT_SZ, INIT_SZ)])

    @pl.when(s == NS - 1)
    def _():
        pltpu.sync_copy(acc.at[pl.ds((NS - 1) * INIT_SZ, INIT_LAST)],
                        agg_hbm.at[pl.ds(base_node + (NS - 1) * INIT_SZ, INIT_LAST)])


_sc_aggregate = functools.partial(
    pl.kernel,
    out_type=jax.ShapeDtypeStruct((N, C), jnp.float32),
    mesh=plsc.VectorSubcoreMesh(core_axis_name="c", subcore_axis_name="s"),
    compiler_params=pltpu.CompilerParams(use_tc_tiling_on_sc=False),
    scratch_types=[
        pltpu.VMEM_SHARED((ACC_ROWS, C), jnp.float32),
        [pltpu.VMEM((CHUNK, C), jnp.float32)] * NBUF,
        pltpu.VMEM((2, IDXBLK), jnp.int32),
        pltpu.VMEM((2, IDXBLK), jnp.int32),
        pltpu.VMEM((2, CPB, CHUNK), jnp.int32),
        pltpu.VMEM((TAIL,), jnp.int32),
        pltpu.SemaphoreType.DMA,
        [pltpu.SemaphoreType.DMA] * NBUF,
        [pltpu.SemaphoreType.DMA] * NBUF,
    ],
)(_sc_body)


def _tc_body(norm_ref, agg_ref, w_ref, out_ref):
    h = norm_ref[...] * agg_ref[...]
    out_ref[...] = jnp.dot(h, w_ref[...], preferred_element_type=jnp.float32)


def _tc_matmul(norm, agg, W):
    return pl.pallas_call(
        _tc_body,
        grid=(N // ROWBLK,),
        in_specs=[
            pl.BlockSpec((ROWBLK, 1), lambda i: (i, 0)),
            pl.BlockSpec((ROWBLK, C), lambda i: (i, 0)),
            pl.BlockSpec((C, C), lambda i: (0, 0)),
        ],
        out_specs=pl.BlockSpec((ROWBLK, C), lambda i: (i, 0)),
        out_shape=jax.ShapeDtypeStruct((N, C), jnp.float32),
    )(norm, agg, W)


def kernel(x, sources, targets, norm, W):
    src = sources.astype(jnp.int32)
    tgt = targets.astype(jnp.int32)
    agg = _sc_aggregate(x, src, tgt)
    return _tc_matmul(norm, agg, W)


# speedup vs baseline: 11.8363x; 1.0576x over previous
import functools

import jax
import jax.numpy as jnp
from jax import lax
from jax.experimental import pallas as pl
from jax.experimental.pallas import tpu as pltpu
from jax.experimental.pallas import tpu_sc as plsc

N = 50000
E = 800000
C = 64

NC = 2
NS = 16
HALF = N // NC
ACC_ROWS = HALF + NS

EPS = E // NS
CHUNK = 96
CPB = 4
IDXBLK = CPB * CHUNK
NBLK = EPS // IDXBLK
TAIL = EPS - NBLK * IDXBLK
CCAP = 496
DUMP = 480

INIT_SZ = 1568
INIT_LAST = HALF - (NS - 1) * INIT_SZ

ROWBLK = 5000


def _sc_body(x_hbm, src_hbm, tgt_hbm, agg_hbm,
             acc, rowbufs, sblk, tblk, csrc, ctgt, s2d, t2d,
             isem, gsems, ssems):
    c = lax.axis_index("c")
    s = lax.axis_index("s")
    base_node = c * HALF
    dummy = HALF + s

    @pl.when(s < NS - 1)
    def _():
        pltpu.sync_copy(x_hbm.at[pl.ds(base_node + s * INIT_SZ, INIT_SZ)],
                        acc.at[pl.ds(s * INIT_SZ, INIT_SZ)])

    @pl.when(s == NS - 1)
    def _():
        pltpu.sync_copy(x_hbm.at[pl.ds(base_node + (NS - 1) * INIT_SZ, INIT_LAST)],
                        acc.at[pl.ds((NS - 1) * INIT_SZ, INIT_LAST)])

    plsc.subcore_barrier()

    e0 = s * EPS

    def load_idx(p, blk, n):
        eb = e0 + blk * IDXBLK
        pltpu.async_copy(src_hbm.at[pl.ds(eb, n)], sblk.at[p].at[pl.ds(0, n)], isem)
        pltpu.async_copy(tgt_hbm.at[pl.ds(eb, n)], tblk.at[p].at[pl.ds(0, n)], isem)

    def drain_idx(p, n):
        pltpu.make_async_copy(src_hbm.at[pl.ds(e0, n)],
                              sblk.at[p].at[pl.ds(0, n)], isem).wait()
        pltpu.make_async_copy(tgt_hbm.at[pl.ds(e0, n)],
                              tblk.at[p].at[pl.ds(0, n)], isem).wait()

    iota16 = lax.iota(jnp.int32, 16)

    def compact(p, mvec_in, ngroups):
        mvec = mvec_in
        for i in range(ngroups):
            sv = sblk[p, pl.ds(i * 16, 16)]
            t = tblk[p, pl.ds(i * 16, 16)]
            tl = t - base_node
            ok = (tl >= 0) & (tl < HALF)
            key = jnp.where(ok, iota16, iota16 + 16)
            _, sv_c = plsc.sort_key_val(key, sv)
            _, tl_c = plsc.sort_key_val(key, tl)
            pos = mvec + iota16
            plsc.store_scatter(csrc, (pos,), sv_c)
            plsc.store_scatter(ctgt, (pos,), tl_c)
            mvec = mvec + plsc.all_reduce_population_count(ok)
        return mvec

    def fire_gather(k):
        return pltpu.async_copy(
            x_hbm.at[s2d.at[k]], rowbufs[k], gsems[k])

    def wait_gather(k):
        pltpu.make_async_copy(
            x_hbm.at[s2d.at[k]], rowbufs[k], gsems[k]).wait()

    def fire_scatter(k):
        pltpu.async_copy(rowbufs[k], acc.at[t2d.at[k]], ssems[k], add=True)

    def drain_scatter(k):
        pltpu.make_async_copy(rowbufs[k], acc.at[t2d.at[k]], ssems[k]).wait()

    def fire_block(m_tot, pend):
        nfire = m_tot // CHUNK
        for k in range(CPB):
            @pl.when(k < nfire)
            def _():
                @pl.when(((pend >> k) & 1) == 1)
                def _():
                    drain_scatter(k)
                for ii in range(CHUNK // 16):
                    s2d[k, pl.ds(ii * 16, 16)] = csrc[pl.ds(k * CHUNK + ii * 16, 16)]
                    t2d[k, pl.ds(ii * 16, 16)] = ctgt[pl.ds(k * CHUNK + ii * 16, 16)]
                fire_gather(k)

        mrem = m_tot - nfire * CHUNK

        @pl.when(nfire > 0)
        def _():
            for i in range(CHUNK // 16):
                @pl.when(i * 16 < mrem)
                def _():
                    src_pos = nfire * CHUNK + i * 16 + iota16
                    csrc[pl.ds(i * 16, 16)] = plsc.load_gather(csrc, (src_pos,))
                    ctgt[pl.ds(i * 16, 16)] = plsc.load_gather(ctgt, (src_pos,))

        pend_out = pend & ~((jnp.int32(1) << nfire) - 1)
        return (nfire.astype(jnp.int32), mrem.astype(jnp.int32),
                pend_out.astype(jnp.int32))

    def scatter_block(nprev, pend):
        for k in range(CPB):
            @pl.when(k < nprev)
            def _():
                wait_gather(k)
                fire_scatter(k)
        return (pend | ((jnp.int32(1) << nprev) - 1)).astype(jnp.int32)

    pltpu.sync_copy(src_hbm.at[pl.ds(e0, IDXBLK)], sblk.at[0])
    pltpu.sync_copy(tgt_hbm.at[pl.ds(e0, IDXBLK)], tblk.at[0])
    load_idx(1, 1, IDXBLK)
    mvec = compact(0, jnp.zeros((16,), jnp.int32), IDXBLK // 16)
    nfire, _, pend = fire_block(jnp.max(mvec), jnp.int32(0))
    mvec = mvec - nfire * CHUNK

    def body(g, carry):
        mvec, nprev, pend = carry
        p = g % 2
        q = 1 - p
        drain_idx(p, IDXBLK)

        @pl.when(g + 1 < NBLK)
        def _():
            load_idx(q, g + 1, IDXBLK)

        pend = scatter_block(nprev, pend)
        mvec = compact(p, mvec, IDXBLK // 16)
        nfire, _, pend = fire_block(jnp.max(mvec), pend)
        return mvec - nfire * CHUNK, nfire, pend

    mvec, nprev, pend = lax.fori_loop(1, NBLK, body, (mvec, nfire, pend))

    pend = scatter_block(nprev, pend)

    et = e0 + NBLK * IDXBLK
    pltpu.sync_copy(src_hbm.at[pl.ds(et, TAIL)], sblk.at[0].at[pl.ds(0, TAIL)])
    pltpu.sync_copy(tgt_hbm.at[pl.ds(et, TAIL)], tblk.at[0].at[pl.ds(0, TAIL)])
    mvec = compact(0, mvec, TAIL // 16)
    m_tot = jnp.max(mvec)
    nfire2 = (m_tot + CHUNK - 1) // CHUNK
    for i in range((2 * CHUNK) // 16):
        lane = lax.iota(jnp.int32, 16) + (i * 16)
        inside = lane < m_tot
        csrc[pl.ds(i * 16, 16)] = jnp.where(inside, csrc[pl.ds(i * 16, 16)], 0)
        ctgt[pl.ds(i * 16, 16)] = jnp.where(inside, ctgt[pl.ds(i * 16, 16)], dummy)
    for k in range(2):
        @pl.when(((pend >> k) & 1) == 1)
        def _():
            drain_scatter(k)
    pend = pend & ~3
    for k in range(2):
        @pl.when(k < nfire2)
        def _():
            for ii in range(CHUNK // 16):
                s2d[k, pl.ds(ii * 16, 16)] = csrc[pl.ds(k * CHUNK + ii * 16, 16)]
                t2d[k, pl.ds(ii * 16, 16)] = ctgt[pl.ds(k * CHUNK + ii * 16, 16)]
            fire_gather(k)
    for k in range(2):
        @pl.when(k < nfire2)
        def _():
            wait_gather(k)
            fire_scatter(k)
    pend = pend | ((jnp.int32(1) << nfire2) - 1)

    for k in range(CPB):
        @pl.when(((pend >> k) & 1) == 1)
        def _():
            drain_scatter(k)

    plsc.subcore_barrier()

    @pl.when(s < NS - 1)
    def _():
        pltpu.sync_copy(acc.at[pl.ds(s * INIT_SZ, INIT_SZ)],
                        agg_hbm.at[pl.ds(base_node + s * INIT_SZ, INIT_SZ)])

    @pl.when(s == NS - 1)
    def _():
        pltpu.sync_copy(acc.at[pl.ds((NS - 1) * INIT_SZ, INIT_LAST)],
                        agg_hbm.at[pl.ds(base_node + (NS - 1) * INIT_SZ, INIT_LAST)])


_sc_aggregate = functools.partial(
    pl.kernel,
    out_type=jax.ShapeDtypeStruct((N, C), jnp.float32),
    mesh=plsc.VectorSubcoreMesh(core_axis_name="c", subcore_axis_name="s"),
    compiler_params=pltpu.CompilerParams(use_tc_tiling_on_sc=False,
                                         needs_layout_passes=False),
    scratch_types=[
        pltpu.VMEM_SHARED((ACC_ROWS, C), jnp.float32),
        [pltpu.VMEM((CHUNK, C), jnp.float32)] * CPB,
        pltpu.VMEM((2, IDXBLK), jnp.int32),
        pltpu.VMEM((2, IDXBLK), jnp.int32),
        pltpu.VMEM((CCAP,), jnp.int32),
        pltpu.VMEM((CCAP,), jnp.int32),
        pltpu.VMEM((CPB, CHUNK), jnp.int32),
        pltpu.VMEM((CPB, CHUNK), jnp.int32),
        pltpu.SemaphoreType.DMA,
        [pltpu.SemaphoreType.DMA] * CPB,
        [pltpu.SemaphoreType.DMA] * CPB,
    ],
)(_sc_body)


def _tc_body(norm_ref, agg_ref, w_ref, out_ref):
    h = norm_ref[...] * agg_ref[...]
    out_ref[...] = jnp.dot(h, w_ref[...], preferred_element_type=jnp.float32)


def _tc_matmul(norm, agg, W):
    return pl.pallas_call(
        _tc_body,
        grid=(N // ROWBLK,),
        in_specs=[
            pl.BlockSpec((ROWBLK, 1), lambda i: (i, 0)),
            pl.BlockSpec((ROWBLK, C), lambda i: (i, 0)),
            pl.BlockSpec((C, C), lambda i: (0, 0)),
        ],
        out_specs=pl.BlockSpec((ROWBLK, C), lambda i: (i, 0)),
        out_shape=jax.ShapeDtypeStruct((N, C), jnp.float32),
    )(norm, agg, W)


def kernel(x, sources, targets, norm, W):
    src = sources.astype(jnp.int32)
    tgt = targets.astype(jnp.int32)
    agg = _sc_aggregate(x, src, tgt)
    return _tc_matmul(norm, agg, W)
